# baseline (speedup 1.0000x reference)
import functools

import jax
import jax.numpy as jnp
from jax import lax
from jax.experimental import pallas as pl
from jax.experimental.pallas import tpu as pltpu
from jax.experimental.pallas import tpu_sc as plsc

_VOCAB = 1_000_000
_D = 64
_NIDX = 20

_WROWS = _VOCAB // 2
_BW = 4096
_BV = 2 * _BW
_NB = (_WROWS + _BW - 1) // _BW
_NEG = -1e30


def _sc_gather_sum(idx, emb):
  mesh = plsc.VectorSubcoreMesh(core_axis_name="c", subcore_axis_name="s")

  @functools.partial(
      pl.kernel,
      mesh=mesh,
      compiler_params=pltpu.CompilerParams(use_tc_tiling_on_sc=False),
      out_type=jax.ShapeDtypeStruct((_D,), jnp.float32),
      scratch_types=[
          pltpu.VMEM((_NIDX,), jnp.int32),
          pltpu.VMEM((_NIDX, _D), jnp.float32),
          pltpu.VMEM((_D,), jnp.float32),
          pltpu.SemaphoreType.DMA,
      ],
  )
  def gather_sum(idx_hbm, emb_hbm, out_hbm, idx_v, rows_v, acc_v, sem):
    wid = lax.axis_index("s") * 2 + lax.axis_index("c")

    @pl.when(wid == 0)
    def _():
      pltpu.sync_copy(idx_hbm, idx_v)
      pltpu.async_copy(emb_hbm.at[idx_v], rows_v, sem).wait()
      for j in range(_D // 16):
        acc = rows_v[0, pl.ds(j * 16, 16)]
        for i in range(1, _NIDX):
          acc = acc + rows_v[i, pl.ds(j * 16, 16)]
        acc_v[pl.ds(j * 16, 16)] = acc
      pltpu.sync_copy(acc_v, out_hbm)

  return gather_sum(idx, emb)


def _fused_body(s2_ref, w_ref, b_ref, out_ref, scratch, m_ref, l_ref):
  p = pl.program_id(0)
  i = pl.program_id(1)

  @pl.when(jnp.logical_and(p == 0, i == 0))
  def _():
    m_ref[...] = jnp.full((1, 128), _NEG, jnp.float32)
    l_ref[...] = jnp.zeros((1, 128), jnp.float32)

  @pl.when(p == 0)
  def _():
    w = w_ref[...]
    ve = lax.dot_general(
        s2_ref[0:1], w, (((1,), (1,)), ((), ())),
        preferred_element_type=jnp.float32)
    vo = lax.dot_general(
        s2_ref[1:2], w, (((1,), (1,)), ((), ())),
        preferred_element_type=jnp.float32)
    inter = jnp.concatenate([ve[:, :, None], vo[:, :, None]], axis=2)
    logits = inter.reshape(1, _BV) + b_ref[...].reshape(1, _BV)
    vidx = lax.broadcasted_iota(jnp.int32, (1, _BV), 1) + i * _BV
    logits = jnp.where(vidx < _VOCAB, logits, _NEG)
    scratch[:, pl.ds(i * _BV, _BV)] = logits
    m_old = m_ref[...]
    bmax = jnp.max(logits, axis=1, keepdims=True)
    m_new = jnp.maximum(m_old, bmax)
    corr = jnp.exp(m_old - m_new)
    bsum = jnp.sum(jnp.exp(logits - m_new[:, 0:1]), axis=1, keepdims=True)
    l_ref[...] = l_ref[...] * corr + bsum
    m_ref[...] = m_new

  @pl.when(p == 1)
  def _():
    logz = m_ref[...] + jnp.log(l_ref[...])
    out_ref[...] = scratch[:, pl.ds(i * _BV, _BV)] - logz[:, 0:1]


def kernel(inputs, emb, W, b):
  idx = inputs.astype(jnp.int32)
  s = _sc_gather_sum(idx, emb).reshape(1, _D)
  s2 = jnp.concatenate(
      [jnp.pad(s, ((0, 0), (0, _D))), jnp.pad(s, ((0, 0), (_D, 0)))], axis=0)
  w2 = W.reshape(_WROWS, 2 * _D)

  out = pl.pallas_call(
      _fused_body,
      grid=(2, _NB),
      in_specs=[
          pl.BlockSpec((2, 2 * _D), lambda p, i: (0, 0)),
          pl.BlockSpec((_BW, 2 * _D), lambda p, i: (i * (1 - p), 0)),
          pl.BlockSpec((_BV,), lambda p, i: (i * (1 - p),)),
      ],
      out_specs=pl.BlockSpec((1, _BV), lambda p, i: (0, i * p)),
      out_shape=jax.ShapeDtypeStruct((1, _VOCAB), jnp.float32),
      scratch_shapes=[
          pltpu.VMEM((1, _NB * _BV), jnp.float32),
          pltpu.VMEM((1, 128), jnp.float32),
          pltpu.VMEM((1, 128), jnp.float32),
      ],
  )(s2, w2, b)

  return out


# speedup vs baseline: 2.0222x; 2.0222x over previous
import jax
import jax.numpy as jnp
from jax import lax
from jax.experimental import pallas as pl
from jax.experimental.pallas import tpu as pltpu

_VOCAB = 1_000_000
_D = 64
_NIDX = 20

_BV = 8192
_NB = (_VOCAB + _BV - 1) // _BV
_NEG = -1e30


def _fused_body(idx_ref, emb_ref, w_ref, b_ref, out_ref,
                scratch, s_ref, rows_ref, m_ref, l_ref, sem):
  p = pl.program_id(0)
  i = pl.program_id(1)

  @pl.when(jnp.logical_and(p == 0, i == 0))
  def _():
    for k in range(_NIDX):
      pltpu.make_async_copy(
          emb_ref.at[pl.ds(idx_ref[k], 1), :], rows_ref.at[pl.ds(k, 1), :], sem
      ).start()
    for k in range(_NIDX):
      pltpu.make_async_copy(
          emb_ref.at[pl.ds(idx_ref[k], 1), :], rows_ref.at[pl.ds(k, 1), :], sem
      ).wait()
    s_ref[...] = jnp.sum(rows_ref[...], axis=0, keepdims=True)
    m_ref[...] = jnp.full((1, 128), _NEG, jnp.float32)
    l_ref[...] = jnp.zeros((1, 128), jnp.float32)

  @pl.when(p == 0)
  def _():
    logits = lax.dot_general(
        s_ref[...], w_ref[...], (((1,), (1,)), ((), ())),
        preferred_element_type=jnp.float32,
    ) + b_ref[...].reshape(1, _BV)
    vidx = lax.broadcasted_iota(jnp.int32, (1, _BV), 1) + i * _BV
    logits = jnp.where(vidx < _VOCAB, logits, _NEG)
    scratch[:, pl.ds(i * _BV, _BV)] = logits
    m_old = m_ref[...]
    bmax = jnp.max(logits, axis=1, keepdims=True)
    m_new = jnp.maximum(m_old, bmax)
    corr = jnp.exp(m_old - m_new)
    bsum = jnp.sum(jnp.exp(logits - m_new[:, 0:1]), axis=1, keepdims=True)
    l_ref[...] = l_ref[...] * corr + bsum
    m_ref[...] = m_new

  @pl.when(p == 1)
  def _():
    logz = m_ref[...] + jnp.log(l_ref[...])
    res = scratch[:, pl.ds(i * _BV, _BV)] - logz[:, 0:1]
    out_ref[...] = res.reshape(_BV)


def kernel(inputs, emb, W, b):
  idx = inputs.astype(jnp.int32)

  out = pl.pallas_call(
      _fused_body,
      grid=(2, _NB),
      in_specs=[
          pl.BlockSpec(memory_space=pltpu.SMEM),
          pl.BlockSpec(memory_space=pltpu.MemorySpace.HBM),
          pl.BlockSpec((_BV, _D), lambda p, i: (i * (1 - p), 0)),
          pl.BlockSpec((_BV,), lambda p, i: (i * (1 - p),)),
      ],
      out_specs=pl.BlockSpec((_BV,), lambda p, i: (i * p,)),
      out_shape=jax.ShapeDtypeStruct((_VOCAB,), jnp.float32),
      scratch_shapes=[
          pltpu.VMEM((1, _NB * _BV), jnp.float32),
          pltpu.VMEM((1, _D), jnp.float32),
          pltpu.VMEM((_NIDX, _D), jnp.float32),
          pltpu.VMEM((1, 128), jnp.float32),
          pltpu.VMEM((1, 128), jnp.float32),
          pltpu.SemaphoreType.DMA,
      ],
  )(idx, emb, W, b)

  return out.reshape(1, _VOCAB)


# speedup vs baseline: 3.2802x; 1.6220x over previous
import jax
import jax.numpy as jnp
from jax import lax
from jax.experimental import pallas as pl
from jax.experimental.pallas import tpu as pltpu

_VOCAB = 1_000_000
_D = 64
_NIDX = 20

_BV = 16384
_NB = (_VOCAB + _BV - 1) // _BV
_NEG = -1e30


def _fused_body(rows_ref, w_ref, b_ref, out_ref, scratch, s_ref, m_ref, l_ref):
  p = pl.program_id(0)
  i = pl.program_id(1)

  @pl.when(jnp.logical_and(p == 0, i == 0))
  def _():
    s_ref[...] = jnp.sum(rows_ref[...], axis=0, keepdims=True)
    m_ref[...] = jnp.full((1, 128), _NEG, jnp.float32)
    l_ref[...] = jnp.zeros((1, 128), jnp.float32)

  @pl.when(p == 0)
  def _():
    logits = lax.dot_general(
        s_ref[...], w_ref[...], (((1,), (1,)), ((), ())),
        preferred_element_type=jnp.float32,
    ) + b_ref[...].reshape(1, _BV)
    vidx = lax.broadcasted_iota(jnp.int32, (1, _BV), 1) + i * _BV
    logits = jnp.where(vidx < _VOCAB, logits, _NEG)
    scratch[:, pl.ds(i * _BV, _BV)] = logits
    m_old = m_ref[...]
    bmax = jnp.max(logits, axis=1, keepdims=True)
    m_new = jnp.maximum(m_old, bmax)
    corr = jnp.exp(m_old - m_new)
    bsum = jnp.sum(jnp.exp(logits - m_new[:, 0:1]), axis=1, keepdims=True)
    l_ref[...] = l_ref[...] * corr + bsum
    m_ref[...] = m_new

  @pl.when(p == 1)
  def _():
    logz = m_ref[...] + jnp.log(l_ref[...])
    res = scratch[:, pl.ds(i * _BV, _BV)] - logz[:, 0:1]
    out_ref[...] = res.reshape(_BV)


def kernel(inputs, emb, W, b):
  idx = inputs.astype(jnp.int32)
  rows = jnp.concatenate(
      [lax.dynamic_slice_in_dim(emb, idx[k], 1, 0) for k in range(_NIDX)],
      axis=0)

  out = pl.pallas_call(
      _fused_body,
      grid=(2, _NB),
      in_specs=[
          pl.BlockSpec((_NIDX, _D), lambda p, i: (0, 0)),
          pl.BlockSpec((_BV, _D), lambda p, i: (i * (1 - p), 0)),
          pl.BlockSpec((_BV,), lambda p, i: (i * (1 - p),)),
      ],
      out_specs=pl.BlockSpec((_BV,), lambda p, i: (i * p,)),
      out_shape=jax.ShapeDtypeStruct((_VOCAB,), jnp.float32),
      scratch_shapes=[
          pltpu.VMEM((1, _NB * _BV), jnp.float32),
          pltpu.VMEM((1, _D), jnp.float32),
          pltpu.VMEM((1, 128), jnp.float32),
          pltpu.VMEM((1, 128), jnp.float32),
      ],
      compiler_params=pltpu.CompilerParams(
          vmem_limit_bytes=110 * 1024 * 1024,
      ),
  )(rows, W, b)

  return out.reshape(1, _VOCAB)


# speedup vs baseline: 5.5923x; 1.7049x over previous
import jax
import jax.numpy as jnp
from jax import lax
from jax.experimental import pallas as pl
from jax.experimental.pallas import tpu as pltpu

_VOCAB = 1_000_000
_D = 64
_NIDX = 20

_BV = 16384
_NB = (_VOCAB + _BV - 1) // _BV
_NEG = -1e30


def _fused_body(rows_ref, w_ref, b_ref, out_ref, scratch, s_ref, m_ref, l_ref):
  p = pl.program_id(0)
  i = pl.program_id(1)

  @pl.when(jnp.logical_and(p == 0, i == 0))
  def _():
    s_ref[...] = jnp.sum(rows_ref[...], axis=0, keepdims=True)
    m_ref[...] = jnp.full((1, 128), _NEG, jnp.float32)
    l_ref[...] = jnp.zeros((1, 128), jnp.float32)

  @pl.when(p == 0)
  def _():
    logits = lax.dot_general(
        s_ref[...], w_ref[...], (((1,), (0,)), ((), ())),
        preferred_element_type=jnp.float32,
    ) + b_ref[...].reshape(1, _BV)
    vidx = lax.broadcasted_iota(jnp.int32, (1, _BV), 1) + i * _BV
    logits = jnp.where(vidx < _VOCAB, logits, _NEG)
    scratch[:, pl.ds(i * _BV, _BV)] = logits
    m_old = m_ref[...]
    bmax = jnp.max(logits, axis=1, keepdims=True)
    m_new = jnp.maximum(m_old, bmax)
    corr = jnp.exp(m_old - m_new)
    bsum = jnp.sum(jnp.exp(logits - m_new[:, 0:1]), axis=1, keepdims=True)
    l_ref[...] = l_ref[...] * corr + bsum
    m_ref[...] = m_new

  @pl.when(p == 1)
  def _():
    logz = m_ref[...] + jnp.log(l_ref[...])
    res = scratch[:, pl.ds(i * _BV, _BV)] - logz[:, 0:1]
    out_ref[...] = res.reshape(_BV)


def kernel(inputs, emb, W, b):
  idx = inputs.astype(jnp.int32)
  rows = jnp.concatenate(
      [lax.dynamic_slice_in_dim(emb, idx[k], 1, 0) for k in range(_NIDX)],
      axis=0)
  z = rows[0, 0] * 0.0
  wt = W.T + z

  out = pl.pallas_call(
      _fused_body,
      grid=(2, _NB),
      in_specs=[
          pl.BlockSpec((_NIDX, _D), lambda p, i: (0, 0)),
          pl.BlockSpec((_D, _BV), lambda p, i: (0, i * (1 - p))),
          pl.BlockSpec((_BV,), lambda p, i: (i * (1 - p),)),
      ],
      out_specs=pl.BlockSpec((_BV,), lambda p, i: (i * p,)),
      out_shape=jax.ShapeDtypeStruct((_VOCAB,), jnp.float32),
      scratch_shapes=[
          pltpu.VMEM((1, _NB * _BV), jnp.float32),
          pltpu.VMEM((1, _D), jnp.float32),
          pltpu.VMEM((1, 128), jnp.float32),
          pltpu.VMEM((1, 128), jnp.float32),
      ],
      compiler_params=pltpu.CompilerParams(
          vmem_limit_bytes=110 * 1024 * 1024,
      ),
  )(rows, wt, b)

  return out.reshape(1, _VOCAB)


# speedup vs baseline: 6.1093x; 1.0924x over previous
import jax
import jax.numpy as jnp
from jax import lax
from jax.experimental import pallas as pl
from jax.experimental.pallas import tpu as pltpu

_VOCAB = 1_000_000
_D = 64
_NIDX = 20

_BV = 32768
_NB = (_VOCAB + _BV - 1) // _BV
_NEG = -1e30


def _fused_body(rows_ref, w_ref, b_ref, out_ref, scratch, s_ref, m_ref, l_ref):
  p = pl.program_id(0)
  i = pl.program_id(1)

  @pl.when(jnp.logical_and(p == 0, i == 0))
  def _():
    s_ref[...] = jnp.sum(rows_ref[...], axis=0, keepdims=True)
    m_ref[...] = jnp.full((1, 128), _NEG, jnp.float32)
    l_ref[...] = jnp.zeros((1, 128), jnp.float32)

  @pl.when(p == 0)
  def _():
    logits = lax.dot_general(
        s_ref[...], w_ref[...], (((1,), (0,)), ((), ())),
        preferred_element_type=jnp.float32,
    ) + b_ref[...].reshape(1, _BV)
    vidx = lax.broadcasted_iota(jnp.int32, (1, _BV), 1) + i * _BV
    logits = jnp.where(vidx < _VOCAB, logits, _NEG)
    scratch[:, pl.ds(i * _BV, _BV)] = logits
    m_old = m_ref[...]
    bmax = jnp.max(logits, axis=1, keepdims=True)
    m_new = jnp.maximum(m_old, bmax)
    corr = jnp.exp(m_old - m_new)
    bsum = jnp.sum(jnp.exp(logits - m_new[:, 0:1]), axis=1, keepdims=True)
    l_ref[...] = l_ref[...] * corr + bsum
    m_ref[...] = m_new

  @pl.when(p == 1)
  def _():
    logz = m_ref[...] + jnp.log(l_ref[...])
    res = scratch[:, pl.ds(i * _BV, _BV)] - logz[:, 0:1]
    out_ref[...] = res.reshape(_BV)


def kernel(inputs, emb, W, b):
  idx = inputs.astype(jnp.int32)
  rows = jnp.concatenate(
      [lax.dynamic_slice_in_dim(emb, idx[k], 1, 0) for k in range(_NIDX)],
      axis=0)
  z = rows[0, 0] * 0.0
  wt = W.T + z

  out = pl.pallas_call(
      _fused_body,
      grid=(2, _NB),
      in_specs=[
          pl.BlockSpec((_NIDX, _D), lambda p, i: (0, 0)),
          pl.BlockSpec((_D, _BV), lambda p, i: (0, i * (1 - p))),
          pl.BlockSpec((_BV,), lambda p, i: (i * (1 - p),)),
      ],
      out_specs=pl.BlockSpec((_BV,), lambda p, i: (i * p,)),
      out_shape=jax.ShapeDtypeStruct((_VOCAB,), jnp.float32),
      scratch_shapes=[
          pltpu.VMEM((1, _NB * _BV), jnp.float32),
          pltpu.VMEM((1, _D), jnp.float32),
          pltpu.VMEM((1, 128), jnp.float32),
          pltpu.VMEM((1, 128), jnp.float32),
      ],
      compiler_params=pltpu.CompilerParams(
          vmem_limit_bytes=110 * 1024 * 1024,
      ),
  )(rows, wt, b)

  return out.reshape(1, _VOCAB)


# speedup vs baseline: 6.1348x; 1.0042x over previous
import jax
import jax.numpy as jnp
from jax import lax
from jax.experimental import pallas as pl
from jax.experimental.pallas import tpu as pltpu

_VOCAB = 1_000_000
_D = 64
_NIDX = 20

_BV = 65536
_NB = (_VOCAB + _BV - 1) // _BV
_NEG = -1e30


def _fused_body(rows_ref, w_ref, b_ref, out_ref, scratch, s_ref, m_ref, l_ref):
  p = pl.program_id(0)
  i = pl.program_id(1)

  @pl.when(jnp.logical_and(p == 0, i == 0))
  def _():
    s_ref[...] = jnp.sum(rows_ref[...], axis=0, keepdims=True)
    m_ref[...] = jnp.full((1, 128), _NEG, jnp.float32)
    l_ref[...] = jnp.zeros((1, 128), jnp.float32)

  @pl.when(p == 0)
  def _():
    logits = lax.dot_general(
        s_ref[...], w_ref[...], (((1,), (0,)), ((), ())),
        preferred_element_type=jnp.float32,
    ) + b_ref[...].reshape(1, _BV)
    vidx = lax.broadcasted_iota(jnp.int32, (1, _BV), 1) + i * _BV
    logits = jnp.where(vidx < _VOCAB, logits, _NEG)
    scratch[:, pl.ds(i * _BV, _BV)] = logits
    m_old = m_ref[...]
    bmax = jnp.max(logits, axis=1, keepdims=True)
    m_new = jnp.maximum(m_old, bmax)
    corr = jnp.exp(m_old - m_new)
    bsum = jnp.sum(jnp.exp(logits - m_new[:, 0:1]), axis=1, keepdims=True)
    l_ref[...] = l_ref[...] * corr + bsum
    m_ref[...] = m_new

  @pl.when(p == 1)
  def _():
    logz = m_ref[...] + jnp.log(l_ref[...])
    res = scratch[:, pl.ds(i * _BV, _BV)] - logz[:, 0:1]
    out_ref[...] = res.reshape(_BV)


def kernel(inputs, emb, W, b):
  idx = inputs.astype(jnp.int32)
  rows = jnp.concatenate(
      [lax.dynamic_slice_in_dim(emb, idx[k], 1, 0) for k in range(_NIDX)],
      axis=0)
  z = rows[0, 0] * 0.0
  wt = W.T + z

  out = pl.pallas_call(
      _fused_body,
      grid=(2, _NB),
      in_specs=[
          pl.BlockSpec((_NIDX, _D), lambda p, i: (0, 0)),
          pl.BlockSpec((_D, _BV), lambda p, i: (0, i * (1 - p))),
          pl.BlockSpec((_BV,), lambda p, i: (i * (1 - p),)),
      ],
      out_specs=pl.BlockSpec((_BV,), lambda p, i: (i * p,)),
      out_shape=jax.ShapeDtypeStruct((_VOCAB,), jnp.float32),
      scratch_shapes=[
          pltpu.VMEM((1, _NB * _BV), jnp.float32),
          pltpu.VMEM((1, _D), jnp.float32),
          pltpu.VMEM((1, 128), jnp.float32),
          pltpu.VMEM((1, 128), jnp.float32),
      ],
      compiler_params=pltpu.CompilerParams(
          vmem_limit_bytes=110 * 1024 * 1024,
      ),
  )(rows, wt, b)

  return out.reshape(1, _VOCAB)
